# baseline (speedup 1.0000x reference)
import functools

import jax
import jax.numpy as jnp
from jax import lax
from jax.experimental import pallas as pl
from jax.experimental.pallas import tpu as pltpu
from jax.experimental.pallas import tpu_sc as plsc

_D = 32
_E = 524288
_NC = 2
_NS = 16
_NW = _NC * _NS
_EPW = _E // _NW
_CK = 256
_NCH = _EPW // _CK
_NB = 2
_L = 16
_EU = 8


def _body(psrc, pdst, nsrc, ndst, uemb, iemb, pos_out, neg_out, outv, *rest):
  bufu = rest[0:_NB]
  bufv = rest[_NB:2 * _NB]
  idxu = rest[2 * _NB:3 * _NB]
  idxv = rest[3 * _NB:4 * _NB]
  semu = rest[4 * _NB:5 * _NB]
  semv = rest[5 * _NB:6 * _NB]
  siu = rest[6 * _NB:7 * _NB]
  siv = rest[7 * _NB:8 * _NB]
  w = lax.axis_index("s") * _NC + lax.axis_index("c")
  lanes = lax.iota(jnp.int32, _L)

  uemb_w = uemb
  iemb_w = iemb

  for src_h, dst_h, out_h in ((psrc, pdst, pos_out), (nsrc, ndst, neg_out)):
    for b in range(_NB):
      pltpu.async_copy(src_h.at[w, b], idxu[b], siu[b])
      pltpu.async_copy(dst_h.at[w, b], idxv[b], siv[b])
    for b in range(_NB):
      pltpu.make_async_copy(src_h.at[w, b], idxu[b], siu[b]).wait()
      pltpu.make_async_copy(dst_h.at[w, b], idxv[b], siv[b]).wait()
      pltpu.async_copy(uemb_w.at[idxu[b]], bufu[b], semu[b])
      pltpu.async_copy(iemb_w.at[idxv[b]], bufv[b], semv[b])

    @pl.loop(0, _NCH, step=_NB)
    def _chunks(gb):
      for b in range(_NB):
        g = gb + b
        ng = g + _NB
        pltpu.make_async_copy(uemb_w.at[idxu[b]], bufu[b], semu[b]).wait()
        pltpu.make_async_copy(iemb_w.at[idxv[b]], bufv[b], semv[b]).wait()

        @pl.when(ng < _NCH)
        def _():
          pltpu.async_copy(src_h.at[w, ng], idxu[b], siu[b])
          pltpu.async_copy(dst_h.at[w, ng], idxv[b], siv[b])

        @pl.loop(0, _CK // _L)
        def _groups(j):
          rows = j * _L + lanes
          acc = jnp.zeros((_L,), jnp.float32)
          colv = jnp.zeros((_L,), jnp.int32)
          for d in range(_D // 2):
            wu = plsc.load_gather(bufu[b], [rows, colv])
            wv = plsc.load_gather(bufv[b], [rows, colv])
            u0, u1 = plsc.unpack(plsc.bitcast(wu, jnp.bfloat16),
                                 format=plsc.PackFormat.INTERLEAVED)
            v0, v1 = plsc.unpack(plsc.bitcast(wv, jnp.bfloat16),
                                 format=plsc.PackFormat.INTERLEAVED)
            acc = acc + u0 * v0 + u1 * v1
            if d < _D // 2 - 1:
              colv = colv + 1
          outv[pl.ds(g * _CK + j * _L, _L)] = acc

        @pl.when(ng < _NCH)
        def _():
          pltpu.make_async_copy(src_h.at[w, ng], idxu[b], siu[b]).wait()
          pltpu.make_async_copy(dst_h.at[w, ng], idxv[b], siv[b]).wait()
          pltpu.async_copy(uemb_w.at[idxu[b]], bufu[b], semu[b])
          pltpu.async_copy(iemb_w.at[idxv[b]], bufv[b], semv[b])

    pltpu.sync_copy(outv, out_h.at[pl.ds(w * _EPW, _EPW)])


@jax.jit
def _scores(psrc, pdst, nsrc, ndst, uemb, iemb):
  mesh = plsc.VectorSubcoreMesh(
      core_axis_name="c", subcore_axis_name="s",
      num_cores=_NC, num_subcores=_NS)
  return pl.kernel(
      _body,
      out_type=(jax.ShapeDtypeStruct((_E,), jnp.float32),
                jax.ShapeDtypeStruct((_E,), jnp.float32)),
      mesh=mesh,
      scratch_types=[
          pltpu.VMEM((_EPW,), jnp.float32),
      ] + [pltpu.VMEM((_CK, _D // 2), jnp.int32) for _ in range(2 * _NB)]
        + [pltpu.VMEM((_CK,), jnp.int32) for _ in range(2 * _NB)]
        + [pltpu.SemaphoreType.DMA for _ in range(4 * _NB)],
      compiler_params=pltpu.CompilerParams(
          use_tc_tiling_on_sc=False, needs_layout_passes=False),
      name="mf_edge_scores",
  )(psrc, pdst, nsrc, ndst, uemb, iemb)


def _pack_rows(x):
  u = jax.lax.bitcast_convert_type(x.astype(jnp.bfloat16), jnp.uint16)
  lo = u[:, 0::2].astype(jnp.uint32)
  hi = u[:, 1::2].astype(jnp.uint32)
  return jax.lax.bitcast_convert_type(lo | (hi << 16), jnp.int32)


def kernel(pos_src, pos_dst, neg_src, neg_dst, user_emb, item_emb):
  ps = pos_src.reshape(_NW, _NCH, _CK)
  pd = pos_dst.reshape(_NW, _NCH, _CK)
  ns = neg_src.reshape(_NW, _NCH, _CK)
  nd = neg_dst.reshape(_NW, _NCH, _CK)
  pos_score, neg_score = _scores(ps, pd, ns, nd,
                                 _pack_rows(user_emb), _pack_rows(item_emb))
  return pos_score.reshape(_E, 1), neg_score.reshape(_E, 1)


# speedup vs baseline: 6.5275x; 6.5275x over previous
import functools

import jax
import jax.numpy as jnp
from jax import lax
from jax.experimental import pallas as pl
from jax.experimental.pallas import tpu as pltpu
from jax.experimental.pallas import tpu_sc as plsc

_D = 32
_E = 524288
_NC = 2
_NS = 16
_NW = _NC * _NS
_EPW = _E // _NW
_CK = 256
_NCH = _EPW // _CK
_NB = 2
_L = 16
_EU = 8


def _body(psrc, pdst, nsrc, ndst, uemb, iemb, pos_out, neg_out, outv, *rest):
  bufu = rest[0:_NB]
  bufv = rest[_NB:2 * _NB]
  idxu = rest[2 * _NB:3 * _NB]
  idxv = rest[3 * _NB:4 * _NB]
  semu = rest[4 * _NB:5 * _NB]
  semv = rest[5 * _NB:6 * _NB]
  siu = rest[6 * _NB:7 * _NB]
  siv = rest[7 * _NB:8 * _NB]
  w = lax.axis_index("s") * _NC + lax.axis_index("c")
  lanes = lax.iota(jnp.int32, _L)

  uemb_w = uemb
  iemb_w = iemb

  for src_h, dst_h, out_h in ((psrc, pdst, pos_out), (nsrc, ndst, neg_out)):
    for b in range(_NB):
      pltpu.async_copy(src_h.at[w, b], idxu[b], siu[b])
      pltpu.async_copy(dst_h.at[w, b], idxv[b], siv[b])
    for b in range(_NB):
      pltpu.make_async_copy(src_h.at[w, b], idxu[b], siu[b]).wait()
      pltpu.make_async_copy(dst_h.at[w, b], idxv[b], siv[b]).wait()
      pltpu.async_copy(uemb_w.at[idxu[b]], bufu[b], semu[b])
      pltpu.async_copy(iemb_w.at[idxv[b]], bufv[b], semv[b])

    @pl.loop(0, _NCH, step=_NB)
    def _chunks(gb):
      for b in range(_NB):
        g = gb + b
        ng = g + _NB
        pltpu.make_async_copy(uemb_w.at[idxu[b]], bufu[b], semu[b]).wait()
        pltpu.make_async_copy(iemb_w.at[idxv[b]], bufv[b], semv[b]).wait()

        @pl.when(ng < _NCH)
        def _():
          pltpu.async_copy(src_h.at[w, ng], idxu[b], siu[b])
          pltpu.async_copy(dst_h.at[w, ng], idxv[b], siv[b])

        @pl.loop(0, _CK // _L)
        def _groups(j):
          rows = j * _L + lanes
          acc = jnp.zeros((_L,), jnp.float32)
          colv = jnp.zeros((_L,), jnp.int32)
          for d in range(_D // 2):
            wu = plsc.load_gather(bufu[b], [rows, colv])
            wv = plsc.load_gather(bufv[b], [rows, colv])
            u0, u1 = plsc.unpack(plsc.bitcast(wu, jnp.bfloat16),
                                 format=plsc.PackFormat.INTERLEAVED)
            v0, v1 = plsc.unpack(plsc.bitcast(wv, jnp.bfloat16),
                                 format=plsc.PackFormat.INTERLEAVED)
            acc = acc + u0 * v0 + u1 * v1
            if d < _D // 2 - 1:
              colv = colv + 1
          outv[pl.ds(g * _CK + j * _L, _L)] = acc

        @pl.when(ng < _NCH)
        def _():
          pltpu.make_async_copy(src_h.at[w, ng], idxu[b], siu[b]).wait()
          pltpu.make_async_copy(dst_h.at[w, ng], idxv[b], siv[b]).wait()
          pltpu.async_copy(uemb_w.at[idxu[b]], bufu[b], semu[b])
          pltpu.async_copy(iemb_w.at[idxv[b]], bufv[b], semv[b])

    pltpu.sync_copy(outv, out_h.at[pl.ds(w * _EPW, _EPW)])


@jax.jit
def _scores(psrc, pdst, nsrc, ndst, uemb, iemb):
  mesh = plsc.VectorSubcoreMesh(
      core_axis_name="c", subcore_axis_name="s",
      num_cores=_NC, num_subcores=_NS)
  return pl.kernel(
      _body,
      out_type=(jax.ShapeDtypeStruct((_E,), jnp.float32),
                jax.ShapeDtypeStruct((_E,), jnp.float32)),
      mesh=mesh,
      scratch_types=[
          pltpu.VMEM((_EPW,), jnp.float32),
      ] + [pltpu.VMEM((_CK, _D // 2), jnp.int32) for _ in range(2 * _NB)]
        + [pltpu.VMEM((_CK,), jnp.int32) for _ in range(2 * _NB)]
        + [pltpu.SemaphoreType.DMA for _ in range(4 * _NB)],
      compiler_params=pltpu.CompilerParams(
          use_tc_tiling_on_sc=False, needs_layout_passes=False),
      name="mf_edge_scores",
  )(psrc, pdst, nsrc, ndst, uemb, iemb)


def _pack_rows(x):
  u = jax.lax.bitcast_convert_type(x.astype(jnp.bfloat16), jnp.uint16)
  lo = u[:, :16].astype(jnp.uint32)
  hi = u[:, 16:].astype(jnp.uint32)
  return jax.lax.bitcast_convert_type(lo | (hi << 16), jnp.int32)


def kernel(pos_src, pos_dst, neg_src, neg_dst, user_emb, item_emb):
  ps = pos_src.reshape(_NW, _NCH, _CK)
  pd = pos_dst.reshape(_NW, _NCH, _CK)
  ns = neg_src.reshape(_NW, _NCH, _CK)
  nd = neg_dst.reshape(_NW, _NCH, _CK)
  pos_score, neg_score = _scores(ps, pd, ns, nd,
                                 _pack_rows(user_emb), _pack_rows(item_emb))
  return pos_score.reshape(_E, 1), neg_score.reshape(_E, 1)
